# Initial kernel scaffold; baseline (speedup 1.0000x reference)
import jax
import jax.numpy as jnp
from jax.experimental import pallas as pl


def kernel(img):
    raise NotImplementedError("write your pallas kernel here")


# speedup vs baseline: 1.3775x; 1.3775x over previous
import jax
import jax.numpy as jnp
import numpy as np
from jax.experimental import pallas as pl
from jax.experimental.pallas import tpu as pltpu

_PATCH = 16
_NPS = 32


def _patch_keep_mask() -> np.ndarray:
    num_patch = _NPS * _NPS
    num_masked = num_patch // 2
    base = jnp.concatenate([
        jnp.ones((num_masked,), jnp.float32),
        jnp.zeros((num_patch - num_masked,), jnp.float32),
    ])
    perm = jax.random.permutation(jax.random.key(42), num_patch)
    masked = np.asarray(base[perm]).reshape(_NPS, _NPS)
    return 1.0 - masked


_KEEP_PATCH = _patch_keep_mask()
_KEEP_FULL = np.kron(_KEEP_PATCH, np.ones((_PATCH, _PATCH), np.float32))


def _body(mask_ref, img_ref, out_ref):
    out_ref[...] = img_ref[...] * mask_ref[...][None, :, :]


def kernel(img):
    c, h, w = img.shape
    cb = 8
    mask = jnp.asarray(_KEEP_FULL)
    return pl.pallas_call(
        _body,
        grid=(c // cb,),
        in_specs=[
            pl.BlockSpec((h, w), lambda i: (0, 0)),
            pl.BlockSpec((cb, h, w), lambda i: (i, 0, 0)),
        ],
        out_specs=pl.BlockSpec((cb, h, w), lambda i: (i, 0, 0)),
        out_shape=jax.ShapeDtypeStruct((c, h, w), img.dtype),
    )(mask, img)
